# baseline (speedup 1.0000x reference)
import functools

import jax
import jax.numpy as jnp
from jax import lax
from jax.experimental import pallas as pl
from jax.experimental.pallas import tpu as pltpu
from jax.experimental.pallas import tpu_sc as plsc

N = 10000
E = 320000
F_IN = 128
EMB = 256
HALF = 128
B = 64
C = 10

NP = 10240
TPB = NP // 16
CH = 80
EPT = E // 16
NCH = EPT // CH
NB = 4
CHD = 400
EPW = E // 32

_f32 = jnp.float32
_mesh = plsc.VectorSubcoreMesh(core_axis_name="c", subcore_axis_name="s")
_HIGH = lax.Precision.HIGHEST


def _dot(a, b):
    return jnp.dot(a, b, preferred_element_type=_f32, precision=_HIGH)


@functools.partial(
    pl.kernel,
    out_type=jax.ShapeDtypeStruct((2, NP), _f32),
    mesh=_mesh,
    scratch_types=[
        pltpu.VMEM((CHD,), jnp.int32),
        pltpu.VMEM((CHD,), _f32),
        pltpu.VMEM((TPB,), _f32),
        pltpu.VMEM_SHARED((NP,), _f32),
        pltpu.SemaphoreType.DMA,
    ],
)
def _sc_degree(dst, degp, dstb, ones, zb, deg_s, sem):
    del sem
    c = lax.axis_index("c")
    s = lax.axis_index("s")
    zv = jnp.zeros((16,), _f32)
    ov = jnp.ones((16,), _f32)
    for j in range(TPB // 16):
        zb[pl.ds(j * 16, 16)] = zv
    for j in range(CHD // 16):
        ones[pl.ds(j * 16, 16)] = ov
    pltpu.sync_copy(zb, deg_s.at[pl.ds(s * TPB, TPB)])
    plsc.subcore_barrier()

    base = (s * 2 + c) * EPW

    def body(i, _):
        off = base + i * CHD
        pltpu.sync_copy(dst.at[pl.ds(off, CHD)], dstb)
        pltpu.sync_copy(ones, deg_s.at[dstb], add=True)
        return 0

    lax.fori_loop(0, EPW // CHD, body, 0)
    plsc.subcore_barrier()
    pltpu.sync_copy(deg_s.at[pl.ds(s * TPB, TPB)], degp.at[c, pl.ds(s * TPB, TPB)])


@functools.partial(
    pl.kernel,
    out_type=(
        jax.ShapeDtypeStruct((NP, HALF), _f32),
        jax.ShapeDtypeStruct((NP, HALF), _f32),
    ),
    mesh=_mesh,
    scratch_types=(
        [pltpu.VMEM((CH,), jnp.int32)] * (2 * NB)
        + [pltpu.VMEM((CH, HALF), _f32)] * NB
        + [pltpu.VMEM_SHARED((NP, HALF), _f32)]
        + [pltpu.SemaphoreType.DMA] * (4 * NB)
    ),
)
def _sc_msgpass(src, dst, hs0, hs1, acc0, acc1, *scr):
    srcbs = scr[0:NB]
    dstbs = scr[NB:2 * NB]
    stages = scr[2 * NB:3 * NB]
    acc_s = scr[3 * NB]
    gsems = scr[3 * NB + 1:3 * NB + 1 + NB]
    ssems = scr[3 * NB + 1 + NB:3 * NB + 1 + 2 * NB]
    isems = scr[3 * NB + 1 + 2 * NB:3 * NB + 1 + 3 * NB]
    dsems = scr[3 * NB + 1 + 3 * NB:3 * NB + 1 + 4 * NB]
    c = lax.axis_index("c")
    s = lax.axis_index("s")
    zv = jnp.zeros((16,), _f32)
    for r in range(CH):
        for j in range(HALF // 16):
            stages[0][r, pl.ds(j * 16, 16)] = zv
    for t in range(TPB // CH):
        pltpu.sync_copy(stages[0], acc_s.at[pl.ds(s * TPB + t * CH, CH)])
    plsc.subcore_barrier()

    def edge_loop(hs_ref):
        base = s * EPT

        def start_src(j, b):
            pltpu.async_copy(src.at[pl.ds(base + j * CH, CH)], srcbs[b],
                             isems[b])

        def start_dst(j, b):
            pltpu.async_copy(dst.at[pl.ds(base + j * CH, CH)], dstbs[b],
                             dsems[b])

        def wait_src(b):
            pltpu.make_async_copy(src.at[pl.ds(base, CH)], srcbs[b],
                                  isems[b]).wait()

        def wait_dst(b):
            pltpu.make_async_copy(dst.at[pl.ds(base, CH)], dstbs[b],
                                  dsems[b]).wait()

        def start_gather(b):
            pltpu.async_copy(hs_ref.at[srcbs[b]], stages[b], gsems[b])

        def start_scatter(b):
            pltpu.async_copy(stages[b], acc_s.at[dstbs[b]], ssems[b])

        def wait_gather(b):
            pltpu.make_async_copy(hs_ref.at[srcbs[b]], stages[b],
                                  gsems[b]).wait()

        def wait_scatter(b):
            pltpu.make_async_copy(stages[b], acc_s.at[dstbs[b]],
                                  ssems[b]).wait()

        for b in range(NB):
            start_src(b, b)
        for b in range(2):
            start_dst(b, b)
            wait_src(b)
            start_gather(b)

        def body(k, _):
            for b in range(NB):
                j = k * NB + b
                bg = (b + 2) % NB

                @pl.when(j < NCH)
                def _():
                    wait_gather(b)
                    wait_dst(b)
                    start_scatter(b)

                @pl.when(j + NB < NCH)
                def _():
                    start_src(j + NB, b)

                @pl.when(j >= 2)
                def _():
                    wait_scatter(bg)

                @pl.when(j + 2 < NCH)
                def _():
                    start_dst(j + 2, bg)
                    wait_src(bg)
                    start_gather(bg)

            return 0

        lax.fori_loop(0, (NCH + 2 + NB - 1) // NB, body, 0)

    @pl.when(c == 0)
    def _():
        edge_loop(hs0)

    @pl.when(c == 1)
    def _():
        edge_loop(hs1)

    plsc.subcore_barrier()

    @pl.when(c == 0)
    def _():
        pltpu.sync_copy(acc_s.at[pl.ds(s * TPB, TPB)], acc0.at[pl.ds(s * TPB, TPB)])

    @pl.when(c == 1)
    def _():
        pltpu.sync_copy(acc_s.at[pl.ds(s * TPB, TPB)], acc1.at[pl.ds(s * TPB, TPB)])


RB = 2048
NG = NP // RB


def _dinv_blk(degp_blk):
    deg = degp_blk[0:1, :] + degp_blk[1:2, :] + 1.0
    return jnp.reshape(lax.rsqrt(deg), (RB, 1))


def _tc_first_body(x, w0, degp, o0, o1):
    hs = _dot(x[...], w0[...]) * _dinv_blk(degp[...])
    o0[...] = hs[:, :HALF]
    o1[...] = hs[:, HALF:]


def _tc_mid_body(a0, a1, h0, h1, degp, bprev, w, o0, o1):
    dinv = _dinv_blk(degp[...])
    b = bprev[...]
    p0 = jax.nn.relu((a0[...] + h0[...]) * dinv + b[:, :HALF])
    p1 = jax.nn.relu((a1[...] + h1[...]) * dinv + b[:, HALF:])
    hs = _dot(jnp.concatenate([p0, p1], axis=1), w[...]) * dinv
    o0[...] = hs[:, :HALF]
    o1[...] = hs[:, HALF:]


def _tc_pool_body(a0, a1, h0, h1, degp, b3, bi, gsum, gmax, cnt):
    i = pl.program_id(0)
    dinv = _dinv_blk(degp[...])
    b = b3[...]
    p0 = jax.nn.relu((a0[...] + h0[...]) * dinv + b[:, :HALF])
    p1 = jax.nn.relu((a1[...] + h1[...]) * dinv + b[:, HALF:])
    h4 = jnp.concatenate([p0, p1], axis=1)

    row = i * RB + lax.broadcasted_iota(jnp.int32, (1, RB), 1)
    valid = row < N
    biv = jnp.where(valid, bi[...], -1)
    oh = (lax.broadcasted_iota(jnp.int32, (B, RB), 0) == biv).astype(_f32)
    ninf = _f32(-jnp.inf)

    @pl.when(i == 0)
    def _():
        gsum[...] = jnp.zeros((B, EMB), _f32)
        gmax[...] = jnp.full((B, EMB), ninf, _f32)
        cnt[...] = jnp.zeros((8, B), _f32)

    gsum[...] += _dot(oh, h4)
    cnt[0:1, :] += jnp.sum(oh, axis=1)[None, :]

    bcol = biv[0, :][:, None]

    def body(bb, _):
        m = jnp.max(jnp.where(bcol == bb, h4, ninf), axis=0)
        gmax[pl.ds(bb, 1), :] = jnp.maximum(gmax[pl.ds(bb, 1), :], m[None, :])
        return 0

    lax.fori_loop(0, B, body, 0)


def _tc_mlp_body(gsum, gmax, cnt, wn1, bn1, wn2, bn2, wn3, bn3, wo, bo, out):
    gmean = gsum[...] / jnp.maximum(cnt[0:1, :], 1.0).reshape(B, 1)
    g = jnp.concatenate([gmean, gmax[...]], axis=1)
    g = jax.nn.relu(_dot(g, wn1[...]) + bn1[...])
    g = jax.nn.relu(_dot(g, wn2[...]) + bn2[...])
    g = jax.nn.relu(_dot(g, wn3[...]) + bn3[...])
    out[...] = _dot(g, wo[...]) + bo[...]


_hs_sds = (jax.ShapeDtypeStruct((NP, HALF), _f32),
           jax.ShapeDtypeStruct((NP, HALF), _f32))

_row_spec = pl.BlockSpec((RB, HALF), lambda i: (i, 0))
_degp_spec = pl.BlockSpec((2, RB), lambda i: (0, i))
_b_spec = pl.BlockSpec((1, EMB), lambda i: (0, 0))
_w_spec = pl.BlockSpec((EMB, EMB), lambda i: (0, 0))
_hs_specs = [_row_spec, _row_spec]

_tc_first = pl.pallas_call(
    _tc_first_body,
    grid=(NG,),
    in_specs=[pl.BlockSpec((RB, F_IN), lambda i: (i, 0)),
              pl.BlockSpec((F_IN, EMB), lambda i: (0, 0)), _degp_spec],
    out_specs=_hs_specs,
    out_shape=_hs_sds,
)

_tc_mid = pl.pallas_call(
    _tc_mid_body,
    grid=(NG,),
    in_specs=[_row_spec, _row_spec, _row_spec, _row_spec, _degp_spec,
              _b_spec, _w_spec],
    out_specs=_hs_specs,
    out_shape=_hs_sds,
)

_acc_spec = pl.BlockSpec((B, EMB), lambda i: (0, 0))
_tc_pool = pl.pallas_call(
    _tc_pool_body,
    grid=(NG,),
    in_specs=[_row_spec, _row_spec, _row_spec, _row_spec, _degp_spec,
              _b_spec, pl.BlockSpec((1, RB), lambda i: (0, i))],
    out_specs=[_acc_spec, _acc_spec, pl.BlockSpec((8, B), lambda i: (0, 0))],
    out_shape=(jax.ShapeDtypeStruct((B, EMB), _f32),
               jax.ShapeDtypeStruct((B, EMB), _f32),
               jax.ShapeDtypeStruct((8, B), _f32)),
)

_tc_mlp = pl.pallas_call(
    _tc_mlp_body,
    out_shape=jax.ShapeDtypeStruct((B, C), _f32),
)


def kernel(x, edge_index, batch_index, W0, b0, W1, b1, W2, b2, W3, b3,
           Wn1, bn1, Wn2, bn2, Wn3, bn3, Wo, bo):
    src = edge_index[0]
    dst = edge_index[1]
    degp = _sc_degree(dst)
    hs0, hs1 = _tc_first(x, W0, degp)
    for bprev, w in ((b0, W1), (b1, W2), (b2, W3)):
        a0, a1 = _sc_msgpass(src, dst, hs0, hs1)
        hs0, hs1 = _tc_mid(a0, a1, hs0, hs1, degp, bprev.reshape(1, EMB), w)
    a0, a1 = _sc_msgpass(src, dst, hs0, hs1)
    gsum, gmax, cnt = _tc_pool(a0, a1, hs0, hs1, degp, b3.reshape(1, EMB),
                               batch_index.reshape(1, N).astype(jnp.int32))
    return _tc_mlp(gsum, gmax, cnt,
                   Wn1, bn1.reshape(1, 512), Wn2, bn2.reshape(1, 256),
                   Wn3, bn3.reshape(1, 128), Wo, bo.reshape(1, C))


# speedup vs baseline: 1.0295x; 1.0295x over previous
import functools

import jax
import jax.numpy as jnp
from jax import lax
from jax.experimental import pallas as pl
from jax.experimental.pallas import tpu as pltpu
from jax.experimental.pallas import tpu_sc as plsc

N = 10000
E = 320000
F_IN = 128
EMB = 256
HALF = 128
B = 64
C = 10

NP = 10240
TPB = NP // 16
CH = 160
EPT = E // 16
NCH = EPT // CH
NB = 2
CHD = 400
EPW = E // 32

_f32 = jnp.float32
_mesh = plsc.VectorSubcoreMesh(core_axis_name="c", subcore_axis_name="s")
_HIGH = lax.Precision.HIGHEST


def _dot(a, b):
    return jnp.dot(a, b, preferred_element_type=_f32, precision=_HIGH)


@functools.partial(
    pl.kernel,
    out_type=jax.ShapeDtypeStruct((2, NP), _f32),
    mesh=_mesh,
    scratch_types=[
        pltpu.VMEM((CHD,), jnp.int32),
        pltpu.VMEM((CHD,), _f32),
        pltpu.VMEM((TPB,), _f32),
        pltpu.VMEM_SHARED((NP,), _f32),
        pltpu.SemaphoreType.DMA,
    ],
)
def _sc_degree(dst, degp, dstb, ones, zb, deg_s, sem):
    del sem
    c = lax.axis_index("c")
    s = lax.axis_index("s")
    zv = jnp.zeros((16,), _f32)
    ov = jnp.ones((16,), _f32)
    for j in range(TPB // 16):
        zb[pl.ds(j * 16, 16)] = zv
    for j in range(CHD // 16):
        ones[pl.ds(j * 16, 16)] = ov
    pltpu.sync_copy(zb, deg_s.at[pl.ds(s * TPB, TPB)])
    plsc.subcore_barrier()

    base = (s * 2 + c) * EPW

    def body(i, _):
        off = base + i * CHD
        pltpu.sync_copy(dst.at[pl.ds(off, CHD)], dstb)
        pltpu.sync_copy(ones, deg_s.at[dstb], add=True)
        return 0

    lax.fori_loop(0, EPW // CHD, body, 0)
    plsc.subcore_barrier()
    pltpu.sync_copy(deg_s.at[pl.ds(s * TPB, TPB)], degp.at[c, pl.ds(s * TPB, TPB)])


@functools.partial(
    pl.kernel,
    out_type=(
        jax.ShapeDtypeStruct((NP, HALF), _f32),
        jax.ShapeDtypeStruct((NP, HALF), _f32),
    ),
    mesh=_mesh,
    scratch_types=(
        [pltpu.VMEM((CH,), jnp.int32)] * (2 * NB)
        + [pltpu.VMEM((CH, HALF), _f32)] * NB
        + [pltpu.VMEM_SHARED((NP, HALF), _f32)]
        + [pltpu.SemaphoreType.DMA] * (4 * NB)
    ),
)
def _sc_msgpass(src, dst, hs0, hs1, acc0, acc1, *scr):
    srcbs = scr[0:NB]
    dstbs = scr[NB:2 * NB]
    stages = scr[2 * NB:3 * NB]
    acc_s = scr[3 * NB]
    gsems = scr[3 * NB + 1:3 * NB + 1 + NB]
    ssems = scr[3 * NB + 1 + NB:3 * NB + 1 + 2 * NB]
    isems = scr[3 * NB + 1 + 2 * NB:3 * NB + 1 + 3 * NB]
    dsems = scr[3 * NB + 1 + 3 * NB:3 * NB + 1 + 4 * NB]
    c = lax.axis_index("c")
    s = lax.axis_index("s")
    zv = jnp.zeros((16,), _f32)
    for r in range(CH):
        for j in range(HALF // 16):
            stages[0][r, pl.ds(j * 16, 16)] = zv
    for t in range(TPB // CH):
        pltpu.sync_copy(stages[0], acc_s.at[pl.ds(s * TPB + t * CH, CH)])
    plsc.subcore_barrier()

    def edge_loop(hs_ref):
        base = s * EPT

        def start_src(j, b):
            pltpu.async_copy(src.at[pl.ds(base + j * CH, CH)], srcbs[b],
                             isems[b])

        def start_dst(j, b):
            pltpu.async_copy(dst.at[pl.ds(base + j * CH, CH)], dstbs[b],
                             dsems[b])

        def wait_src(b):
            pltpu.make_async_copy(src.at[pl.ds(base, CH)], srcbs[b],
                                  isems[b]).wait()

        def wait_dst(b):
            pltpu.make_async_copy(dst.at[pl.ds(base, CH)], dstbs[b],
                                  dsems[b]).wait()

        def start_gather(b):
            pltpu.async_copy(hs_ref.at[srcbs[b]], stages[b], gsems[b])

        def start_scatter(b):
            pltpu.async_copy(stages[b], acc_s.at[dstbs[b]], ssems[b])

        def wait_gather(b):
            pltpu.make_async_copy(hs_ref.at[srcbs[b]], stages[b],
                                  gsems[b]).wait()

        def wait_scatter(b):
            pltpu.make_async_copy(stages[b], acc_s.at[dstbs[b]],
                                  ssems[b]).wait()

        del start_scatter, wait_scatter
        for b in range(2):
            start_src(b, b)
            start_dst(b, b)
        for b in range(2):
            wait_src(b)
            start_gather(b)

        def body(k, _):
            for b in range(2):
                j = 2 * k + b
                wait_gather(b)

                @pl.when(j + 2 < NCH)
                def _():
                    start_src(j + 2, b)

                wait_dst(b)
                pltpu.sync_copy(stages[b], acc_s.at[dstbs[b]], add=True)

                @pl.when(j + 2 < NCH)
                def _():
                    start_dst(j + 2, b)
                    wait_src(b)
                    start_gather(b)

            return 0

        lax.fori_loop(0, NCH // 2, body, 0)
        if NCH % 2:
            wait_gather(0)
            wait_dst(0)
            pltpu.sync_copy(stages[0], acc_s.at[dstbs[0]], add=True)

    @pl.when(c == 0)
    def _():
        edge_loop(hs0)

    @pl.when(c == 1)
    def _():
        edge_loop(hs1)

    plsc.subcore_barrier()

    @pl.when(c == 0)
    def _():
        pltpu.sync_copy(acc_s.at[pl.ds(s * TPB, TPB)], acc0.at[pl.ds(s * TPB, TPB)])

    @pl.when(c == 1)
    def _():
        pltpu.sync_copy(acc_s.at[pl.ds(s * TPB, TPB)], acc1.at[pl.ds(s * TPB, TPB)])


RB = 2048
NG = NP // RB


def _dinv_blk(degp_blk):
    deg = degp_blk[0:1, :] + degp_blk[1:2, :] + 1.0
    return jnp.reshape(lax.rsqrt(deg), (RB, 1))


def _tc_first_body(x, w0, degp, o0, o1):
    hs = _dot(x[...], w0[...]) * _dinv_blk(degp[...])
    o0[...] = hs[:, :HALF]
    o1[...] = hs[:, HALF:]


def _tc_mid_body(a0, a1, h0, h1, degp, bprev, w, o0, o1):
    dinv = _dinv_blk(degp[...])
    b = bprev[...]
    p0 = jax.nn.relu((a0[...] + h0[...]) * dinv + b[:, :HALF])
    p1 = jax.nn.relu((a1[...] + h1[...]) * dinv + b[:, HALF:])
    hs = _dot(jnp.concatenate([p0, p1], axis=1), w[...]) * dinv
    o0[...] = hs[:, :HALF]
    o1[...] = hs[:, HALF:]


def _tc_pool_body(a0, a1, h0, h1, degp, b3, bi, gsum, gmax, cnt):
    i = pl.program_id(0)
    dinv = _dinv_blk(degp[...])
    b = b3[...]
    p0 = jax.nn.relu((a0[...] + h0[...]) * dinv + b[:, :HALF])
    p1 = jax.nn.relu((a1[...] + h1[...]) * dinv + b[:, HALF:])
    h4 = jnp.concatenate([p0, p1], axis=1)

    row = i * RB + lax.broadcasted_iota(jnp.int32, (1, RB), 1)
    valid = row < N
    biv = jnp.where(valid, bi[...], -1)
    oh = (lax.broadcasted_iota(jnp.int32, (B, RB), 0) == biv).astype(_f32)
    ninf = _f32(-jnp.inf)

    @pl.when(i == 0)
    def _():
        gsum[...] = jnp.zeros((B, EMB), _f32)
        gmax[...] = jnp.full((B, EMB), ninf, _f32)
        cnt[...] = jnp.zeros((8, B), _f32)

    gsum[...] += _dot(oh, h4)
    cnt[0:1, :] += jnp.sum(oh, axis=1)[None, :]

    bcol = biv[0, :][:, None]

    def body(bb, _):
        m = jnp.max(jnp.where(bcol == bb, h4, ninf), axis=0)
        gmax[pl.ds(bb, 1), :] = jnp.maximum(gmax[pl.ds(bb, 1), :], m[None, :])
        return 0

    lax.fori_loop(0, B, body, 0)


def _tc_mlp_body(gsum, gmax, cnt, wn1, bn1, wn2, bn2, wn3, bn3, wo, bo, out):
    gmean = gsum[...] / jnp.maximum(cnt[0:1, :], 1.0).reshape(B, 1)
    g = jnp.concatenate([gmean, gmax[...]], axis=1)
    g = jax.nn.relu(_dot(g, wn1[...]) + bn1[...])
    g = jax.nn.relu(_dot(g, wn2[...]) + bn2[...])
    g = jax.nn.relu(_dot(g, wn3[...]) + bn3[...])
    out[...] = _dot(g, wo[...]) + bo[...]


_hs_sds = (jax.ShapeDtypeStruct((NP, HALF), _f32),
           jax.ShapeDtypeStruct((NP, HALF), _f32))

_row_spec = pl.BlockSpec((RB, HALF), lambda i: (i, 0))
_degp_spec = pl.BlockSpec((2, RB), lambda i: (0, i))
_b_spec = pl.BlockSpec((1, EMB), lambda i: (0, 0))
_w_spec = pl.BlockSpec((EMB, EMB), lambda i: (0, 0))
_hs_specs = [_row_spec, _row_spec]

_tc_first = pl.pallas_call(
    _tc_first_body,
    grid=(NG,),
    in_specs=[pl.BlockSpec((RB, F_IN), lambda i: (i, 0)),
              pl.BlockSpec((F_IN, EMB), lambda i: (0, 0)), _degp_spec],
    out_specs=_hs_specs,
    out_shape=_hs_sds,
)

_tc_mid = pl.pallas_call(
    _tc_mid_body,
    grid=(NG,),
    in_specs=[_row_spec, _row_spec, _row_spec, _row_spec, _degp_spec,
              _b_spec, _w_spec],
    out_specs=_hs_specs,
    out_shape=_hs_sds,
)

_acc_spec = pl.BlockSpec((B, EMB), lambda i: (0, 0))
_tc_pool = pl.pallas_call(
    _tc_pool_body,
    grid=(NG,),
    in_specs=[_row_spec, _row_spec, _row_spec, _row_spec, _degp_spec,
              _b_spec, pl.BlockSpec((1, RB), lambda i: (0, i))],
    out_specs=[_acc_spec, _acc_spec, pl.BlockSpec((8, B), lambda i: (0, 0))],
    out_shape=(jax.ShapeDtypeStruct((B, EMB), _f32),
               jax.ShapeDtypeStruct((B, EMB), _f32),
               jax.ShapeDtypeStruct((8, B), _f32)),
)

_tc_mlp = pl.pallas_call(
    _tc_mlp_body,
    out_shape=jax.ShapeDtypeStruct((B, C), _f32),
)


def kernel(x, edge_index, batch_index, W0, b0, W1, b1, W2, b2, W3, b3,
           Wn1, bn1, Wn2, bn2, Wn3, bn3, Wo, bo):
    src = edge_index[0]
    dst = edge_index[1]
    degp = _sc_degree(dst)
    hs0, hs1 = _tc_first(x, W0, degp)
    for bprev, w in ((b0, W1), (b1, W2), (b2, W3)):
        a0, a1 = _sc_msgpass(src, dst, hs0, hs1)
        hs0, hs1 = _tc_mid(a0, a1, hs0, hs1, degp, bprev.reshape(1, EMB), w)
    a0, a1 = _sc_msgpass(src, dst, hs0, hs1)
    gsum, gmax, cnt = _tc_pool(a0, a1, hs0, hs1, degp, b3.reshape(1, EMB),
                               batch_index.reshape(1, N).astype(jnp.int32))
    return _tc_mlp(gsum, gmax, cnt,
                   Wn1, bn1.reshape(1, 512), Wn2, bn2.reshape(1, 256),
                   Wn3, bn3.reshape(1, 128), Wo, bo.reshape(1, C))


# speedup vs baseline: 1.1088x; 1.0770x over previous
import functools

import jax
import jax.numpy as jnp
from jax import lax
from jax.experimental import pallas as pl
from jax.experimental.pallas import tpu as pltpu
from jax.experimental.pallas import tpu_sc as plsc

N = 10000
E = 320000
F_IN = 128
EMB = 256
HALF = 128
B = 64
C = 10

NP = 10240
TPB = NP // 16
CH = 160
EPT = E // 16
NCH = EPT // CH
NB = 2
CHD = 400
EPW = E // 32

_f32 = jnp.float32
_mesh = plsc.VectorSubcoreMesh(core_axis_name="c", subcore_axis_name="s")
_HIGH = lax.Precision.HIGHEST


def _dot(a, b):
    return jnp.dot(a, b, preferred_element_type=_f32, precision=_HIGH)


@functools.partial(
    pl.kernel,
    out_type=jax.ShapeDtypeStruct((2, NP), _f32),
    mesh=_mesh,
    scratch_types=[
        pltpu.VMEM((CHD,), jnp.int32),
        pltpu.VMEM((CHD,), _f32),
        pltpu.VMEM((TPB,), _f32),
        pltpu.VMEM_SHARED((NP,), _f32),
        pltpu.SemaphoreType.DMA,
    ],
)
def _sc_degree(dst, degp, dstb, ones, zb, deg_s, sem):
    del sem
    c = lax.axis_index("c")
    s = lax.axis_index("s")
    zv = jnp.zeros((16,), _f32)
    ov = jnp.ones((16,), _f32)
    for j in range(TPB // 16):
        zb[pl.ds(j * 16, 16)] = zv
    for j in range(CHD // 16):
        ones[pl.ds(j * 16, 16)] = ov
    pltpu.sync_copy(zb, deg_s.at[pl.ds(s * TPB, TPB)])
    plsc.subcore_barrier()

    base = (s * 2 + c) * EPW

    def body(i, _):
        off = base + i * CHD
        pltpu.sync_copy(dst.at[pl.ds(off, CHD)], dstb)
        pltpu.sync_copy(ones, deg_s.at[dstb], add=True)
        return 0

    lax.fori_loop(0, EPW // CHD, body, 0)
    plsc.subcore_barrier()
    pltpu.sync_copy(deg_s.at[pl.ds(s * TPB, TPB)], degp.at[c, pl.ds(s * TPB, TPB)])


@functools.partial(
    pl.kernel,
    out_type=(
        jax.ShapeDtypeStruct((NP, HALF), _f32),
        jax.ShapeDtypeStruct((NP, HALF), _f32),
    ),
    mesh=_mesh,
    scratch_types=(
        [pltpu.VMEM((CH,), jnp.int32)] * (2 * NB)
        + [pltpu.VMEM((CH, HALF), _f32)] * NB
        + [pltpu.VMEM_SHARED((NP, HALF), _f32)]
        + [pltpu.SemaphoreType.DMA] * (4 * NB)
    ),
)
def _sc_msgpass(src, dst, hs0, hs1, acc0, acc1, *scr):
    srcbs = scr[0:NB]
    dstbs = scr[NB:2 * NB]
    stages = scr[2 * NB:3 * NB]
    acc_s = scr[3 * NB]
    gsems = scr[3 * NB + 1:3 * NB + 1 + NB]
    ssems = scr[3 * NB + 1 + NB:3 * NB + 1 + 2 * NB]
    isems = scr[3 * NB + 1 + 2 * NB:3 * NB + 1 + 3 * NB]
    dsems = scr[3 * NB + 1 + 3 * NB:3 * NB + 1 + 4 * NB]
    c = lax.axis_index("c")
    s = lax.axis_index("s")
    zv = jnp.zeros((16,), _f32)
    for r in range(CH):
        for j in range(HALF // 16):
            stages[0][r, pl.ds(j * 16, 16)] = zv
    for t in range(TPB // CH):
        pltpu.sync_copy(stages[0], acc_s.at[pl.ds(s * TPB + t * CH, CH)])
    plsc.subcore_barrier()

    def edge_loop(hs_ref):
        base = s * EPT

        def start_src(j, b):
            pltpu.async_copy(src.at[pl.ds(base + j * CH, CH)], srcbs[b],
                             isems[b])

        def start_dst(j, b):
            pltpu.async_copy(dst.at[pl.ds(base + j * CH, CH)], dstbs[b],
                             dsems[b])

        def wait_src(b):
            pltpu.make_async_copy(src.at[pl.ds(base, CH)], srcbs[b],
                                  isems[b]).wait()

        def wait_dst(b):
            pltpu.make_async_copy(dst.at[pl.ds(base, CH)], dstbs[b],
                                  dsems[b]).wait()

        def start_gather(b):
            pltpu.async_copy(hs_ref.at[srcbs[b]], stages[b], gsems[b])

        def start_scatter(b):
            pltpu.async_copy(stages[b], acc_s.at[dstbs[b]], ssems[b])

        def wait_gather(b):
            pltpu.make_async_copy(hs_ref.at[srcbs[b]], stages[b],
                                  gsems[b]).wait()

        def wait_scatter(b):
            pltpu.make_async_copy(stages[b], acc_s.at[dstbs[b]],
                                  ssems[b]).wait()

        del start_scatter, wait_scatter
        for b in range(2):
            start_src(b, b)
            start_dst(b, b)
        for b in range(2):
            wait_src(b)
            start_gather(b)

        def body(k, _):
            for b in range(2):
                j = 2 * k + b
                wait_gather(b)

                @pl.when(j + 2 < NCH)
                def _():
                    start_src(j + 2, b)

                wait_dst(b)
                pltpu.sync_copy(stages[b], acc_s.at[dstbs[b]], add=True)

                @pl.when(j + 2 < NCH)
                def _():
                    start_dst(j + 2, b)
                    wait_src(b)
                    start_gather(b)

            return 0

        lax.fori_loop(0, NCH // 2, body, 0)
        if NCH % 2:
            wait_gather(0)
            wait_dst(0)
            pltpu.sync_copy(stages[0], acc_s.at[dstbs[0]], add=True)

    @pl.when(c == 0)
    def _():
        edge_loop(hs0)

    @pl.when(c == 1)
    def _():
        edge_loop(hs1)

    plsc.subcore_barrier()

    @pl.when(c == 0)
    def _():
        pltpu.sync_copy(acc_s.at[pl.ds(s * TPB, TPB)], acc0.at[pl.ds(s * TPB, TPB)])

    @pl.when(c == 1)
    def _():
        pltpu.sync_copy(acc_s.at[pl.ds(s * TPB, TPB)], acc1.at[pl.ds(s * TPB, TPB)])


RB = 2048
NG = NP // RB


def _dinv_blk(degp_blk):
    deg = degp_blk[0:1, :] + degp_blk[1:2, :] + 1.0
    return jnp.reshape(lax.rsqrt(deg), (RB, 1))


def _tc_first_body(x, w0, degp, o0, o1):
    hs = _dot(x[...], w0[...]) * _dinv_blk(degp[...])
    o0[...] = hs[:, :HALF]
    o1[...] = hs[:, HALF:]


def _tc_mid_body(a0, a1, h0, h1, degp, bprev, w, o0, o1):
    dinv = _dinv_blk(degp[...])
    b = bprev[...]
    p0 = jax.nn.relu((a0[...] + h0[...]) * dinv + b[:, :HALF])
    p1 = jax.nn.relu((a1[...] + h1[...]) * dinv + b[:, HALF:])
    hs = _dot(jnp.concatenate([p0, p1], axis=1), w[...]) * dinv
    o0[...] = hs[:, :HALF]
    o1[...] = hs[:, HALF:]


def _tc_tail_body(a0, a1, h0, h1, degp, b3, bi, wn1, bn1, wn2, bn2, wn3, bn3,
                  wo, bo, out, gsum, gmax, cnt):
    i = pl.program_id(0)
    dinv = _dinv_blk(degp[...])
    b = b3[...]
    p0 = jax.nn.relu((a0[...] + h0[...]) * dinv + b[:, :HALF])
    p1 = jax.nn.relu((a1[...] + h1[...]) * dinv + b[:, HALF:])
    h4 = jnp.concatenate([p0, p1], axis=1)

    row = i * RB + lax.broadcasted_iota(jnp.int32, (1, RB), 1)
    valid = row < N
    biv = jnp.where(valid, bi[...], -1)
    oh = (lax.broadcasted_iota(jnp.int32, (B, RB), 0) == biv).astype(_f32)
    ninf = _f32(-jnp.inf)

    @pl.when(i == 0)
    def _():
        gsum[...] = jnp.zeros((B, EMB), _f32)
        gmax[...] = jnp.full((B, EMB), ninf, _f32)
        cnt[...] = jnp.zeros((8, B), _f32)

    gsum[...] += _dot(oh, h4)
    cnt[0:1, :] += jnp.sum(oh, axis=1)[None, :]

    lo = jnp.min(jnp.where(valid, biv, B - 1))
    hi = jnp.max(biv)
    bcol = biv[0, :][:, None]

    def body(bb, _):
        m = jnp.max(jnp.where(bcol == bb, h4, ninf), axis=0)
        gmax[pl.ds(bb, 1), :] = jnp.maximum(gmax[pl.ds(bb, 1), :], m[None, :])
        return 0

    lax.fori_loop(lo, hi + 1, body, 0)

    @pl.when(i == NG - 1)
    def _():
        gmean = gsum[...] / jnp.maximum(cnt[0:1, :], 1.0).reshape(B, 1)
        g = jnp.concatenate([gmean, gmax[...]], axis=1)
        g = jax.nn.relu(_dot(g, wn1[...]) + bn1[...])
        g = jax.nn.relu(_dot(g, wn2[...]) + bn2[...])
        g = jax.nn.relu(_dot(g, wn3[...]) + bn3[...])
        out[...] = _dot(g, wo[...]) + bo[...]


_hs_sds = (jax.ShapeDtypeStruct((NP, HALF), _f32),
           jax.ShapeDtypeStruct((NP, HALF), _f32))

_row_spec = pl.BlockSpec((RB, HALF), lambda i: (i, 0))
_degp_spec = pl.BlockSpec((2, RB), lambda i: (0, i))
_b_spec = pl.BlockSpec((1, EMB), lambda i: (0, 0))
_w_spec = pl.BlockSpec((EMB, EMB), lambda i: (0, 0))
_hs_specs = [_row_spec, _row_spec]

_tc_first = pl.pallas_call(
    _tc_first_body,
    grid=(NG,),
    in_specs=[pl.BlockSpec((RB, F_IN), lambda i: (i, 0)),
              pl.BlockSpec((F_IN, EMB), lambda i: (0, 0)), _degp_spec],
    out_specs=_hs_specs,
    out_shape=_hs_sds,
)

_tc_mid = pl.pallas_call(
    _tc_mid_body,
    grid=(NG,),
    in_specs=[_row_spec, _row_spec, _row_spec, _row_spec, _degp_spec,
              _b_spec, _w_spec],
    out_specs=_hs_specs,
    out_shape=_hs_sds,
)

def _full(shape):
    return pl.BlockSpec(shape, lambda i: tuple(0 for _ in shape))


_tc_tail = pl.pallas_call(
    _tc_tail_body,
    grid=(NG,),
    in_specs=[_row_spec, _row_spec, _row_spec, _row_spec, _degp_spec,
              _b_spec, pl.BlockSpec((1, RB), lambda i: (0, i)),
              _full((512, 512)), _full((1, 512)), _full((512, 256)),
              _full((1, 256)), _full((256, 128)), _full((1, 128)),
              _full((128, C)), _full((1, C))],
    out_specs=pl.BlockSpec((B, C), lambda i: (0, 0)),
    out_shape=jax.ShapeDtypeStruct((B, C), _f32),
    scratch_shapes=[pltpu.VMEM((B, EMB), _f32), pltpu.VMEM((B, EMB), _f32),
                    pltpu.VMEM((8, B), _f32)],
)


def kernel(x, edge_index, batch_index, W0, b0, W1, b1, W2, b2, W3, b3,
           Wn1, bn1, Wn2, bn2, Wn3, bn3, Wo, bo):
    src = edge_index[0]
    dst = edge_index[1]
    degp = _sc_degree(dst)
    hs0, hs1 = _tc_first(x, W0, degp)
    for bprev, w in ((b0, W1), (b1, W2), (b2, W3)):
        a0, a1 = _sc_msgpass(src, dst, hs0, hs1)
        hs0, hs1 = _tc_mid(a0, a1, hs0, hs1, degp, bprev.reshape(1, EMB), w)
    a0, a1 = _sc_msgpass(src, dst, hs0, hs1)
    return _tc_tail(a0, a1, hs0, hs1, degp, b3.reshape(1, EMB),
                    batch_index.reshape(1, N).astype(jnp.int32),
                    Wn1, bn1.reshape(1, 512), Wn2, bn2.reshape(1, 256),
                    Wn3, bn3.reshape(1, 128), Wo, bo.reshape(1, C))
